# Initial kernel scaffold; baseline (speedup 1.0000x reference)
import jax
import jax.numpy as jnp
from jax import lax
from jax.experimental import pallas as pl
from jax.experimental.pallas import tpu as pltpu
from jax.experimental.pallas import tpu_sc as plsc

B, L, D = 16, 4096, 768
NSEG = 128
LANES = 16
CHUNK = 64
HALF = L // 2
NCHUNK = HALF // CHUNK
DV = D // LANES


def _body(x_hbm, idx_hbm, out_hbm, xbuf, ibuf_all, ibuf2, obuf, zcbuf, cntl,
          acc_sh, cnt_sh):
    c = lax.axis_index("c")
    s = lax.axis_index("s")
    bl = s % 8
    b = c * 8 + bl
    h = s // 8

    zero16 = jnp.zeros((LANES,), jnp.float32)
    one16 = jnp.ones((LANES,), jnp.float32)

    def zero_row(r, _):
        for j in range(DV):
            xbuf[r, pl.ds(j * LANES, LANES)] = zero16
        zcbuf[r, pl.ds(0, LANES)] = zero16
        obuf[r, pl.ds(0, LANES)] = one16
        return 0
    lax.fori_loop(0, CHUNK, zero_row, 0)

    rbase = bl * NSEG + h * 64
    pltpu.sync_copy(xbuf, acc_sh.at[pl.ds(rbase, 64), :])
    pltpu.sync_copy(zcbuf, cnt_sh.at[pl.ds(rbase, 64), :])

    pltpu.sync_copy(idx_hbm.at[b, pl.ds(h * HALF, HALF)], ibuf_all)

    plsc.subcore_barrier()

    seg_off = bl * NSEG
    for i in range(NCHUNK):
        for k in range(CHUNK // LANES):
            ibuf2[pl.ds(k * LANES, LANES)] = (
                ibuf_all[pl.ds(i * CHUNK + k * LANES, LANES)] + seg_off)
        pltpu.sync_copy(x_hbm.at[b, pl.ds(h * HALF + i * CHUNK, CHUNK), :],
                        xbuf)
        pltpu.sync_copy(xbuf, acc_sh.at[ibuf2], add=True)
        pltpu.sync_copy(obuf, cnt_sh.at[ibuf2], add=True)

    plsc.subcore_barrier()

    pltpu.sync_copy(acc_sh.at[pl.ds(rbase, 64), :], xbuf)
    pltpu.sync_copy(cnt_sh.at[pl.ds(rbase, 64), :], cntl)

    def fin_row(r, _):
        cv = cntl[r, pl.ds(0, LANES)]
        rinv = 1.0 / jnp.maximum(cv, 1.0)
        for j in range(DV):
            xbuf[r, pl.ds(j * LANES, LANES)] = (
                xbuf[r, pl.ds(j * LANES, LANES)] * rinv)
        return 0
    lax.fori_loop(0, 64, fin_row, 0)

    pltpu.sync_copy(xbuf, out_hbm.at[b, pl.ds(h * 64, 64), :])


def kernel(x, sentence_idx):
    mesh = plsc.VectorSubcoreMesh(core_axis_name="c", subcore_axis_name="s")
    f = pl.kernel(
        _body,
        out_type=jax.ShapeDtypeStruct((B, NSEG, D), jnp.float32),
        mesh=mesh,
        scratch_types=[
            pltpu.VMEM((CHUNK, D), jnp.float32),
            pltpu.VMEM((HALF,), jnp.int32),
            pltpu.VMEM((CHUNK,), jnp.int32),
            pltpu.VMEM((CHUNK, LANES), jnp.float32),
            pltpu.VMEM((CHUNK, LANES), jnp.float32),
            pltpu.VMEM((CHUNK, LANES), jnp.float32),
            pltpu.VMEM_SHARED((8 * NSEG, D), jnp.float32),
            pltpu.VMEM_SHARED((8 * NSEG, LANES), jnp.float32),
        ],
    )
    return f(x, sentence_idx.astype(jnp.int32))


# speedup vs baseline: 1.6468x; 1.6468x over previous
import jax
import jax.numpy as jnp
from jax import lax
from jax.experimental import pallas as pl
from jax.experimental.pallas import tpu as pltpu
from jax.experimental.pallas import tpu_sc as plsc

B, L, D = 16, 4096, 768
NSEG = 128
LANES = 16
DH = D // 2
JV = DH // LANES
CHUNK = 64
NCHUNK = L // CHUNK


def _body(x_hbm, idx_hbm, out_hbm, xb, ib, acc, cnt):
    c = lax.axis_index("c")
    s = lax.axis_index("s")
    b = c * 8 + s % 8
    dh = s // 8

    zero16 = jnp.zeros((LANES,), jnp.float32)
    one16 = jnp.ones((LANES,), jnp.float32)

    def zero_row(r, _):
        for j in range(JV):
            acc[r, pl.ds(j * LANES, LANES)] = zero16
        cnt[r, pl.ds(0, LANES)] = zero16
        return 0
    lax.fori_loop(0, NSEG, zero_row, 0)

    pltpu.sync_copy(idx_hbm.at[b, :], ib)

    col0 = dh * DH

    def chunk_body(ch, _):
        pltpu.sync_copy(
            x_hbm.at[b, pl.ds(ch * CHUNK, CHUNK), pl.ds(col0, DH)], xb)

        def grp(g, _):
            t0 = ch * CHUNK + g * LANES
            ids = ib[pl.ds(t0, LANES)]
            for l in range(LANES):
                seg = ids[l]
                row = g * LANES + l
                plsc.addupdate(cnt.at[seg, pl.ds(0, LANES)], one16)
                for j in range(JV):
                    v = xb[row, pl.ds(j * LANES, LANES)]
                    plsc.addupdate(acc.at[seg, pl.ds(j * LANES, LANES)], v)
            return 0
        lax.fori_loop(0, CHUNK // LANES, grp, 0)
        return 0
    lax.fori_loop(0, NCHUNK, chunk_body, 0)

    def fin_row(r, _):
        rinv = 1.0 / jnp.maximum(cnt[r, pl.ds(0, LANES)], 1.0)
        for j in range(JV):
            acc[r, pl.ds(j * LANES, LANES)] = (
                acc[r, pl.ds(j * LANES, LANES)] * rinv)
        return 0
    lax.fori_loop(0, NSEG, fin_row, 0)

    pltpu.sync_copy(acc, out_hbm.at[b, :, pl.ds(col0, DH)])


def kernel(x, sentence_idx):
    mesh = plsc.VectorSubcoreMesh(core_axis_name="c", subcore_axis_name="s")
    f = pl.kernel(
        _body,
        out_type=jax.ShapeDtypeStruct((B, NSEG, D), jnp.float32),
        mesh=mesh,
        compiler_params=pltpu.CompilerParams(needs_layout_passes=False),
        scratch_types=[
            pltpu.VMEM((CHUNK, DH), jnp.float32),
            pltpu.VMEM((L,), jnp.int32),
            pltpu.VMEM((NSEG, DH), jnp.float32),
            pltpu.VMEM((NSEG, LANES), jnp.float32),
        ],
    )
    return f(x, sentence_idx.astype(jnp.int32))


# speedup vs baseline: 2.0467x; 1.2429x over previous
import jax
import jax.numpy as jnp
from jax import lax
from jax.experimental import pallas as pl
from jax.experimental.pallas import tpu as pltpu
from jax.experimental.pallas import tpu_sc as plsc

B, L, D = 16, 4096, 768
NSEG = 128
LANES = 16
DH = D // 2
JV = DH // LANES
CHUNK = 64
NCHUNK = L // CHUNK


def _body(x_hbm, idx_hbm, out_hbm, xb0, xb1, ib, acc, cnt, sem0, sem1):
    c = lax.axis_index("c")
    s = lax.axis_index("s")
    b = c * 8 + s % 8
    dh = s // 8

    zero16 = jnp.zeros((LANES,), jnp.float32)
    one16 = jnp.ones((LANES,), jnp.float32)

    col0 = dh * DH

    def xsrc(ch):
        return x_hbm.at[b, pl.ds(ch * CHUNK, CHUNK), pl.ds(col0, DH)]

    pltpu.make_async_copy(xsrc(0), xb0, sem0).start()

    def zero_row(r, _):
        for j in range(JV):
            acc[r, pl.ds(j * LANES, LANES)] = zero16
        cnt[r, pl.ds(0, LANES)] = zero16
        return 0
    lax.fori_loop(0, NSEG, zero_row, 0)

    pltpu.sync_copy(idx_hbm.at[b, :], ib)

    def process(xb, ch):
        def grp(g, _):
            t0 = ch * CHUNK + g * LANES
            ids = ib[pl.ds(t0, LANES)]
            for l in range(LANES):
                seg = ids[l]
                row = g * LANES + l
                plsc.addupdate(cnt.at[seg, pl.ds(0, LANES)], one16)
                for j in range(JV):
                    v = xb[row, pl.ds(j * LANES, LANES)]
                    plsc.addupdate(acc.at[seg, pl.ds(j * LANES, LANES)], v)
            return 0
        lax.fori_loop(0, CHUNK // LANES, grp, 0)

    def pair_body(p, _):
        ch0 = 2 * p
        pltpu.make_async_copy(xsrc(ch0 + 1), xb1, sem1).start()
        pltpu.make_async_copy(xsrc(ch0), xb0, sem0).wait()
        process(xb0, ch0)
        @pl.when(p < NCHUNK // 2 - 1)
        def _():
            pltpu.make_async_copy(xsrc(ch0 + 2), xb0, sem0).start()
        pltpu.make_async_copy(xsrc(ch0 + 1), xb1, sem1).wait()
        process(xb1, ch0 + 1)
        return 0
    lax.fori_loop(0, NCHUNK // 2, pair_body, 0)

    def fin_row(r, _):
        rinv = 1.0 / jnp.maximum(cnt[r, pl.ds(0, LANES)], 1.0)
        for j in range(JV):
            acc[r, pl.ds(j * LANES, LANES)] = (
                acc[r, pl.ds(j * LANES, LANES)] * rinv)
        return 0
    lax.fori_loop(0, NSEG, fin_row, 0)

    pltpu.sync_copy(acc, out_hbm.at[b, :, pl.ds(col0, DH)])


def kernel(x, sentence_idx):
    mesh = plsc.VectorSubcoreMesh(core_axis_name="c", subcore_axis_name="s")
    f = pl.kernel(
        _body,
        out_type=jax.ShapeDtypeStruct((B, NSEG, D), jnp.float32),
        mesh=mesh,
        compiler_params=pltpu.CompilerParams(needs_layout_passes=False),
        scratch_types=[
            pltpu.VMEM((CHUNK, DH), jnp.float32),
            pltpu.VMEM((CHUNK, DH), jnp.float32),
            pltpu.VMEM((L,), jnp.int32),
            pltpu.VMEM((NSEG, DH), jnp.float32),
            pltpu.VMEM((NSEG, LANES), jnp.float32),
            pltpu.SemaphoreType.DMA,
            pltpu.SemaphoreType.DMA,
        ],
    )
    return f(x, sentence_idx.astype(jnp.int32))


# speedup vs baseline: 4.3140x; 2.1078x over previous
import jax
import jax.numpy as jnp
from jax import lax
from jax.experimental import pallas as pl
from jax.experimental.pallas import tpu as pltpu
from jax.experimental.pallas import tpu_sc as plsc

B, L, D = 16, 4096, 768
NSEG = 128
LANES = 16
DH = D // 2
JV = DH // LANES
CHUNK = 64
NCHUNK = L // CHUNK


def _body(x_hbm, idx_hbm, out_hbm, xb0, xb1, ib, acc, cnt, sem0, sem1):
    c = lax.axis_index("c")
    s = lax.axis_index("s")
    b = c * 8 + s % 8
    dh = s // 8

    zero16 = jnp.zeros((LANES,), jnp.float32)
    one16 = jnp.ones((LANES,), jnp.float32)

    col0 = dh * DH

    def xsrc(ch):
        return x_hbm.at[b, pl.ds(ch * CHUNK, CHUNK), pl.ds(col0, DH)]

    pltpu.make_async_copy(xsrc(0), xb0, sem0).start()

    def zero_row(r, _):
        for j in range(JV):
            acc[r, pl.ds(j * LANES, LANES)] = zero16
        cnt[r, pl.ds(0, LANES)] = zero16
        return 0
    lax.fori_loop(0, NSEG, zero_row, 0)

    pltpu.sync_copy(idx_hbm.at[b, :], ib)

    def process(xb, ch):
        def grp(g, _):
            t0 = ch * CHUNK + g * LANES
            ids = ib[pl.ds(t0, LANES)]
            for l in range(LANES):
                seg = ids[l]
                row = g * LANES + l
                plsc.addupdate(cnt.at[seg, pl.ds(0, LANES)], one16)
                vs = [xb[row, pl.ds(j * LANES, LANES)] for j in range(JV)]
                for j in range(JV):
                    plsc.addupdate(acc.at[seg, pl.ds(j * LANES, LANES)],
                                   vs[j])
            return 0
        lax.fori_loop(0, CHUNK // LANES, grp, 0)

    def pair_body(p, _):
        ch0 = 2 * p
        pltpu.make_async_copy(xsrc(ch0 + 1), xb1, sem1).start()
        pltpu.make_async_copy(xsrc(ch0), xb0, sem0).wait()
        process(xb0, ch0)
        @pl.when(p < NCHUNK // 2 - 1)
        def _():
            pltpu.make_async_copy(xsrc(ch0 + 2), xb0, sem0).start()
        pltpu.make_async_copy(xsrc(ch0 + 1), xb1, sem1).wait()
        process(xb1, ch0 + 1)
        return 0
    lax.fori_loop(0, NCHUNK // 2, pair_body, 0)

    def fin_row(r, _):
        rinv = 1.0 / jnp.maximum(cnt[r, pl.ds(0, LANES)], 1.0)
        for j in range(JV):
            acc[r, pl.ds(j * LANES, LANES)] = (
                acc[r, pl.ds(j * LANES, LANES)] * rinv)
        return 0
    lax.fori_loop(0, NSEG, fin_row, 0)

    pltpu.sync_copy(acc, out_hbm.at[b, :, pl.ds(col0, DH)])


def kernel(x, sentence_idx):
    mesh = plsc.VectorSubcoreMesh(core_axis_name="c", subcore_axis_name="s")
    f = pl.kernel(
        _body,
        out_type=jax.ShapeDtypeStruct((B, NSEG, D), jnp.float32),
        mesh=mesh,
        compiler_params=pltpu.CompilerParams(needs_layout_passes=False),
        scratch_types=[
            pltpu.VMEM((CHUNK, DH), jnp.float32),
            pltpu.VMEM((CHUNK, DH), jnp.float32),
            pltpu.VMEM((L,), jnp.int32),
            pltpu.VMEM((NSEG, DH), jnp.float32),
            pltpu.VMEM((NSEG, LANES), jnp.float32),
            pltpu.SemaphoreType.DMA,
            pltpu.SemaphoreType.DMA,
        ],
    )
    return f(x, sentence_idx.astype(jnp.int32))
